# baseline (speedup 1.0000x reference)
import functools

import jax
import jax.numpy as jnp
from jax import lax
from jax.experimental import pallas as pl
from jax.experimental.pallas import tpu as pltpu
from jax.experimental.pallas import tpu_sc as plsc

_NBUF = 4
_W = 128


def _emb_lookup(weight, idx_t):
    H, B = idx_t.shape
    _, D = weight.shape
    info = plsc.get_sparse_core_info()
    num_cores = info.num_cores
    nw = num_cores * info.num_subcores
    bpw = B // nw
    sub = bpw // _W
    groups = H * sub
    assert groups % _NBUF == 0
    rounds = groups // _NBUF
    nh = _W // 16

    mesh = plsc.VectorSubcoreMesh(core_axis_name="c", subcore_axis_name="s")

    @functools.partial(
        pl.kernel,
        mesh=mesh,
        compiler_params=pltpu.CompilerParams(
            use_tc_tiling_on_sc=False, needs_layout_passes=False),
        out_type=jax.ShapeDtypeStruct((H, D, B), jnp.float32),
        scratch_types=[
            [pltpu.VMEM((_W,), jnp.int32) for _ in range(_NBUF)],
            [pltpu.VMEM((_W, D), jnp.float32) for _ in range(_NBUF)],
            [pltpu.VMEM((D, _W), jnp.float32) for _ in range(_NBUF)],
            [pltpu.SemaphoreType.DMA for _ in range(_NBUF)],
            [pltpu.SemaphoreType.DMA for _ in range(_NBUF)],
        ],
    )
    def emb(w_hbm, idx_hbm, out_hbm, idx_v, rows_v, rowst_v, gsem, ssem):
        wid = lax.axis_index("s") * num_cores + lax.axis_index("c")
        b0 = wid * bpw
        lanes = lax.iota(jnp.int32, 16)
        bvecs = [lanes + 16 * h for h in range(nh)]

        def fire(g, ib):
            h = g // sub
            c0 = b0 + (g % sub) * _W
            pltpu.sync_copy(idx_hbm.at[h, pl.ds(c0, _W)], idx_v[ib])
            pltpu.async_copy(w_hbm.at[idx_v[ib]], rows_v[ib], gsem[ib])

        def wait_gathers(ib):
            pltpu.make_async_copy(
                w_hbm.at[idx_v[ib]], rows_v[ib], gsem[ib]).wait()

        def transpose(ib):
            dlo = lanes
            dhi = lanes + 16

            def bbody(b, carry):
                bvec = lanes * 0 + b
                xlo = rows_v[ib][b, pl.ds(0, 16)]
                xhi = rows_v[ib][b, pl.ds(16, 16)]
                plsc.store_scatter(rowst_v[ib], [dlo, bvec], xlo)
                plsc.store_scatter(rowst_v[ib], [dhi, bvec], xhi)
                return carry

            lax.fori_loop(0, _W, bbody, 0)

        def store(g, ib):
            h = g // sub
            c0 = b0 + (g % sub) * _W
            pltpu.async_copy(
                rowst_v[ib], out_hbm.at[h, :, pl.ds(c0, _W)], ssem[ib])

        def wait_store(ib):
            pltpu.make_async_copy(
                rowst_v[ib], out_hbm.at[0, :, pl.ds(b0, _W)], ssem[ib]).wait()

        for b in range(_NBUF - 1):
            fire(b, b)

        def body(r, carry):
            for b in range(_NBUF):
                g = r * _NBUF + b
                wait_gathers(b)
                bprev = (b - 1) % _NBUF
                gf = g + _NBUF - 1

                @pl.when(gf < groups)
                def _():
                    fire(gf, bprev)

                @pl.when(r > 0)
                def _():
                    wait_store(b)

                transpose(b)
                store(g, b)

            return carry

        lax.fori_loop(0, rounds, body, 0)
        for b in range(_NBUF):
            wait_store(b)

    return emb(weight, idx_t)


def kernel(token_ids, weight):
    out_hdb = _emb_lookup(weight, token_ids.T.astype(jnp.int32))
    return jnp.transpose(out_hdb, (2, 0, 1))


# speedup vs baseline: 1.2306x; 1.2306x over previous
import functools

import jax
import jax.numpy as jnp
from jax import lax
from jax.experimental import pallas as pl
from jax.experimental.pallas import tpu as pltpu
from jax.experimental.pallas import tpu_sc as plsc

_NBUF = 8
_W = 128


def _emb_lookup(weight, idx_t):
    H, B = idx_t.shape
    _, D = weight.shape
    info = plsc.get_sparse_core_info()
    num_cores = info.num_cores
    nw = num_cores * info.num_subcores
    bpw = B // nw
    sub = bpw // _W
    groups = H * sub
    assert groups % _NBUF == 0
    rounds = groups // _NBUF

    mesh = plsc.VectorSubcoreMesh(core_axis_name="c", subcore_axis_name="s")

    @functools.partial(
        pl.kernel,
        mesh=mesh,
        compiler_params=pltpu.CompilerParams(
            use_tc_tiling_on_sc=False, needs_layout_passes=False),
        out_type=jax.ShapeDtypeStruct((B, H, D), jnp.float32),
        scratch_types=[
            [pltpu.VMEM((_W,), jnp.int32) for _ in range(_NBUF)],
            [pltpu.VMEM((_W, D), jnp.float32) for _ in range(_NBUF)],
            [pltpu.SemaphoreType.DMA for _ in range(_NBUF)],
            [pltpu.SemaphoreType.DMA for _ in range(_NBUF)],
        ],
    )
    def emb(w_hbm, idx_hbm, out_hbm, idx_v, rows_v, gsem, ssem):
        wid = lax.axis_index("s") * num_cores + lax.axis_index("c")
        b0 = wid * bpw

        def fire(g, ib):
            h = g // sub
            c0 = b0 + (g % sub) * _W
            pltpu.sync_copy(idx_hbm.at[h, pl.ds(c0, _W)], idx_v[ib])
            pltpu.async_copy(w_hbm.at[idx_v[ib]], rows_v[ib], gsem[ib])

        def wait_gather(ib):
            pltpu.make_async_copy(
                w_hbm.at[idx_v[ib]], rows_v[ib], gsem[ib]).wait()

        def store(g, ib):
            h = g // sub
            c0 = b0 + (g % sub) * _W
            pltpu.async_copy(
                rows_v[ib], out_hbm.at[pl.ds(c0, _W), h, :], ssem[ib])

        def wait_store(ib):
            pltpu.make_async_copy(
                rows_v[ib], out_hbm.at[pl.ds(b0, _W), 0, :], ssem[ib]).wait()

        for b in range(_NBUF - 1):
            fire(b, b)

        def body(r, carry):
            for b in range(_NBUF):
                g = r * _NBUF + b
                wait_gather(b)
                store(g, b)
                ibf = (b - 1) % _NBUF
                gf = g + _NBUF - 1

                @pl.when(gf < groups)
                def _():
                    @pl.when(gf >= _NBUF)
                    def _():
                        wait_store(ibf)

                    fire(gf, ibf)

            return carry

        lax.fori_loop(0, rounds, body, 0)
        for b in range(_NBUF):
            wait_store(b)

    return emb(weight, idx_t)


def kernel(token_ids, weight):
    return _emb_lookup(weight, token_ids.T.astype(jnp.int32))


# speedup vs baseline: 1.3435x; 1.0917x over previous
import functools

import jax
import jax.numpy as jnp
from jax import lax
from jax.experimental import pallas as pl
from jax.experimental.pallas import tpu as pltpu
from jax.experimental.pallas import tpu_sc as plsc

_NBUF = 4
_W = 512


def _emb_lookup(weight, idx_t):
    H, B = idx_t.shape
    _, D = weight.shape
    info = plsc.get_sparse_core_info()
    num_cores = info.num_cores
    nw = num_cores * info.num_subcores
    bpw = B // nw
    sub = bpw // _W
    groups = H * sub
    assert groups % _NBUF == 0
    rounds = groups // _NBUF

    mesh = plsc.VectorSubcoreMesh(core_axis_name="c", subcore_axis_name="s")

    @functools.partial(
        pl.kernel,
        mesh=mesh,
        compiler_params=pltpu.CompilerParams(
            use_tc_tiling_on_sc=False, needs_layout_passes=False),
        out_type=jax.ShapeDtypeStruct((B, H, D), jnp.float32),
        scratch_types=[
            [pltpu.VMEM((_W,), jnp.int32) for _ in range(_NBUF)],
            [pltpu.VMEM((_W, D), jnp.float32) for _ in range(_NBUF)],
            [pltpu.SemaphoreType.DMA for _ in range(_NBUF)],
            [pltpu.SemaphoreType.DMA for _ in range(_NBUF)],
        ],
    )
    def emb(w_hbm, idx_hbm, out_hbm, idx_v, rows_v, gsem, ssem):
        wid = lax.axis_index("s") * num_cores + lax.axis_index("c")
        b0 = wid * bpw

        def fire(g, ib):
            h = g // sub
            c0 = b0 + (g % sub) * _W
            pltpu.sync_copy(idx_hbm.at[h, pl.ds(c0, _W)], idx_v[ib])
            pltpu.async_copy(w_hbm.at[idx_v[ib]], rows_v[ib], gsem[ib])

        def wait_gather(ib):
            pltpu.make_async_copy(
                w_hbm.at[idx_v[ib]], rows_v[ib], gsem[ib]).wait()

        def store(g, ib):
            h = g // sub
            c0 = b0 + (g % sub) * _W
            pltpu.async_copy(
                rows_v[ib], out_hbm.at[pl.ds(c0, _W), h, :], ssem[ib])

        def wait_store(ib):
            pltpu.make_async_copy(
                rows_v[ib], out_hbm.at[pl.ds(b0, _W), 0, :], ssem[ib]).wait()

        for b in range(_NBUF - 1):
            fire(b, b)

        def body(r, carry):
            for b in range(_NBUF):
                g = r * _NBUF + b
                wait_gather(b)
                store(g, b)
                ibf = (b - 1) % _NBUF
                gf = g + _NBUF - 1

                @pl.when(gf < groups)
                def _():
                    @pl.when(gf >= _NBUF)
                    def _():
                        wait_store(ibf)

                    fire(gf, ibf)

            return carry

        lax.fori_loop(0, rounds, body, 0)
        for b in range(_NBUF):
            wait_store(b)

    return emb(weight, idx_t)


def kernel(token_ids, weight):
    return _emb_lookup(weight, token_ids.T.astype(jnp.int32))
